# baseline (speedup 1.0000x reference)
import functools

import jax
import jax.numpy as jnp
from jax import lax
from jax.experimental import pallas as pl
from jax.experimental.pallas import tpu as pltpu
from jax.experimental.pallas import tpu_sc as plsc
from jax._src.pallas import mpmd as pl_mpmd

CODEBOOK_SIZE = 8192
CODEBOOK_DIM = 256
N_TOKENS = 262144

NUM_CORES = 2
NUM_SUBCORES = 16
NUM_WORKERS = NUM_CORES * NUM_SUBCORES
B_PER_W = N_TOKENS // NUM_WORKERS
CHUNK = 64
NCHUNK = B_PER_W // CHUNK
NBUF = 3

_VMESH = plsc.VectorSubcoreMesh(core_axis_name="c", subcore_axis_name="s")
_SMESH = plsc.ScalarSubcoreMesh(axis_name="c", num_cores=2)


def _tec_fn(weight_hbm, idx_hbm, out_hbm, spm):
    s = lax.axis_index("s")
    wid = s * NUM_CORES + lax.axis_index("c")

    def body(idx_v, rows_v, gsem0, gsem1, gsem2, xsem0, xsem1, xsem2):
        gsems = [gsem0, gsem1, gsem2]
        xsems = [xsem0, xsem1, xsem2]
        pltpu.sync_copy(idx_hbm.at[wid], idx_v)

        def start_gather(g, b):
            pltpu.make_async_copy(
                weight_hbm.at[idx_v.at[g]], rows_v.at[b], gsems[b]).start()

        def wait_gather(b):
            pltpu.make_async_copy(
                weight_hbm.at[idx_v.at[0]], rows_v.at[b], gsems[b]).wait()

        def start_xcopy(b):
            pltpu.make_async_copy(rows_v.at[b], spm.at[s, b], xsems[b]).start()

        def wait_xcopy(b):
            pltpu.make_async_copy(rows_v.at[b], spm.at[s, b], xsems[b]).wait()

        start_gather(0, 0)
        start_gather(1, 1)
        wait_gather(0)
        start_xcopy(0)
        start_gather(2, 2)
        wait_gather(1)
        start_xcopy(1)
        wait_xcopy(0)
        start_gather(3, 0)

        def steady(i, carry):
            for j in range(NBUF):
                g = 2 + NBUF * i + j
                b = (2 + j) % NBUF
                pm1 = (1 + j) % NBUF
                wait_gather(b)
                start_xcopy(b)
                wait_xcopy(pm1)
                start_gather(g + 2, pm1)
            return carry

        lax.fori_loop(0, (NCHUNK - 5) // NBUF, steady, 0)

        wait_gather(2)
        start_xcopy(2)
        wait_xcopy(1)
        start_gather(NCHUNK - 1, 1)
        wait_gather(0)
        start_xcopy(0)
        wait_xcopy(2)
        wait_gather(1)
        start_xcopy(1)
        wait_xcopy(0)
        wait_xcopy(1)

    pl.run_scoped(
        body,
        pltpu.VMEM((NCHUNK, CHUNK), jnp.int32),
        pltpu.VMEM((NBUF, CHUNK, CODEBOOK_DIM), jnp.float32),
        pltpu.SemaphoreType.DMA,
        pltpu.SemaphoreType.DMA,
        pltpu.SemaphoreType.DMA,
        pltpu.SemaphoreType.DMA,
        pltpu.SemaphoreType.DMA,
        pltpu.SemaphoreType.DMA,
    )


def _scs_fn(weight_hbm, idx_hbm, out_hbm, spm):
    c = lax.axis_index("c")

    def body(dsem0, dsem1):
        dsems = [dsem0, dsem1]

        def start_round(g, p):
            b = g % NBUF
            for t in range(NUM_SUBCORES):
                base = (t * NUM_CORES + c) * B_PER_W + g * CHUNK
                pltpu.make_async_copy(
                    spm.at[t, b], out_hbm.at[pl.ds(base, CHUNK)],
                    dsems[p]).start()

        def wait_round(p):
            for t in range(NUM_SUBCORES):
                pltpu.make_async_copy(
                    spm.at[t, 0], out_hbm.at[pl.ds(0, CHUNK)],
                    dsems[p]).wait()

        start_round(0, 0)
        start_round(1, 1)

        def outer(i, carry):
            for p in (0, 1):
                g = 2 * i + p
                wait_round(p)
                start_round(g, p)
            return carry

        lax.fori_loop(1, NCHUNK // 2, outer, 0)
        wait_round(0)
        wait_round(1)

    pl.run_scoped(body, pltpu.SemaphoreType.DMA, pltpu.SemaphoreType.DMA)


_mpmd_kernel = pl_mpmd.mpmd_map(
    [(_SMESH, _scs_fn), (_VMESH, _tec_fn)],
    out_types=jax.ShapeDtypeStruct((N_TOKENS, CODEBOOK_DIM), jnp.float32),
    scratch_types=[
        pltpu.VMEM_SHARED((NUM_SUBCORES, NBUF, CHUNK, CODEBOOK_DIM),
                          jnp.float32),
    ],
)


def kernel(embed_id, weight):
    idx = embed_id.astype(jnp.int32).reshape(NUM_WORKERS, NCHUNK, CHUNK)
    return _mpmd_kernel(weight, idx)


# speedup vs baseline: 1.0032x; 1.0032x over previous
import functools

import jax
import jax.numpy as jnp
from jax import lax
from jax.experimental import pallas as pl
from jax.experimental.pallas import tpu as pltpu
from jax.experimental.pallas import tpu_sc as plsc

CODEBOOK_SIZE = 8192
CODEBOOK_DIM = 256
N_TOKENS = 262144

NUM_CORES = 2
NUM_SUBCORES = 16
NUM_WORKERS = NUM_CORES * NUM_SUBCORES
B_PER_W = N_TOKENS // NUM_WORKERS
CHUNK = 64
NCHUNK = B_PER_W // CHUNK

_MESH = plsc.VectorSubcoreMesh(core_axis_name="c", subcore_axis_name="s")


@functools.partial(
    pl.kernel,
    mesh=_MESH,
    out_type=jax.ShapeDtypeStruct((N_TOKENS, CODEBOOK_DIM), jnp.float32),
    scratch_types=[
        pltpu.VMEM((NCHUNK, CHUNK), jnp.int32),
        pltpu.VMEM((3, CHUNK, CODEBOOK_DIM), jnp.float32),
        pltpu.SemaphoreType.DMA,
        pltpu.SemaphoreType.DMA,
        pltpu.SemaphoreType.DMA,
        pltpu.SemaphoreType.DMA,
        pltpu.SemaphoreType.DMA,
        pltpu.SemaphoreType.DMA,
    ],
)
def _codebook_gather(weight_hbm, idx_hbm, out_hbm, idx_v, rows_v,
                     gsem0, gsem1, gsem2, ssem0, ssem1, ssem2):
    wid = lax.axis_index("s") * NUM_CORES + lax.axis_index("c")
    base = wid * B_PER_W
    gsems = [gsem0, gsem1, gsem2]
    ssems = [ssem0, ssem1, ssem2]

    pltpu.sync_copy(idx_hbm.at[wid], idx_v)

    def start_gather(g, buf):
        pltpu.make_async_copy(
            weight_hbm.at[idx_v.at[g]], rows_v.at[buf], gsems[buf]).start()

    def wait_gather(buf):
        pltpu.make_async_copy(
            weight_hbm.at[idx_v.at[0]], rows_v.at[buf], gsems[buf]).wait()

    def start_store(g, buf):
        pltpu.make_async_copy(
            rows_v.at[buf], out_hbm.at[pl.ds(base + g * CHUNK, CHUNK)],
            ssems[buf]).start()

    def wait_store(buf):
        pltpu.make_async_copy(
            rows_v.at[buf], out_hbm.at[pl.ds(base, CHUNK)], ssems[buf]).wait()

    NBUF = 3

    start_gather(0, 0)
    start_gather(1, 1)
    start_gather(2, 2)
    wait_gather(0)
    start_store(0, 0)
    wait_store(0)
    start_gather(3, 0)
    wait_gather(1)
    start_store(1, 1)

    def steady(i, carry):
        for j in range(NBUF):
            g = 2 + NBUF * i + j
            cur = (2 + j) % NBUF
            prv = (1 + j) % NBUF
            wait_store(prv)
            start_gather(g + 2, prv)
            wait_gather(cur)
            start_store(g, cur)
        return carry

    lax.fori_loop(0, (NCHUNK - 5) // NBUF, steady, 0)

    g = NCHUNK - 3
    wait_store((g - 1) % NBUF)
    start_gather(NCHUNK - 1, (g - 1) % NBUF)
    wait_gather(g % NBUF)
    start_store(g, g % NBUF)
    for g in (NCHUNK - 2, NCHUNK - 1):
        wait_gather(g % NBUF)
        start_store(g, g % NBUF)
    for b in range(NBUF):
        wait_store(b)


def kernel(embed_id, weight):
    idx = embed_id.astype(jnp.int32).reshape(NUM_WORKERS, NCHUNK, CHUNK)
    return _codebook_gather(weight, idx)


# speedup vs baseline: 1.0193x; 1.0161x over previous
import functools

import jax
import jax.numpy as jnp
from jax import lax
from jax.experimental import pallas as pl
from jax.experimental.pallas import tpu as pltpu
from jax.experimental.pallas import tpu_sc as plsc

CODEBOOK_SIZE = 8192
CODEBOOK_DIM = 256
N_TOKENS = 262144

NUM_CORES = 2
NUM_SUBCORES = 16
NUM_WORKERS = NUM_CORES * NUM_SUBCORES
B_PER_W = N_TOKENS // NUM_WORKERS
CHUNK = 64
NCHUNK = B_PER_W // CHUNK
NBUF = 3

_MESH = plsc.VectorSubcoreMesh(core_axis_name="c", subcore_axis_name="s")


@functools.partial(
    pl.kernel,
    mesh=_MESH,
    out_type=jax.ShapeDtypeStruct((N_TOKENS, CODEBOOK_DIM), jnp.float32),
    scratch_types=[
        pltpu.VMEM((NCHUNK, CHUNK), jnp.int32),
        pltpu.VMEM((NBUF, CHUNK, CODEBOOK_DIM), jnp.float32),
        pltpu.VMEM_SHARED((NUM_SUBCORES, NBUF, CHUNK, CODEBOOK_DIM),
                          jnp.float32),
        pltpu.SemaphoreType.DMA,
        pltpu.SemaphoreType.DMA,
        pltpu.SemaphoreType.DMA,
        pltpu.SemaphoreType.DMA,
        pltpu.SemaphoreType.DMA,
        pltpu.SemaphoreType.DMA,
        pltpu.SemaphoreType.DMA,
        pltpu.SemaphoreType.DMA,
        pltpu.SemaphoreType.DMA,
    ],
)
def _codebook_gather(weight_hbm, idx_hbm, out_hbm, idx_v, rows_v, spm,
                     gsem0, gsem1, gsem2, xsem0, xsem1, xsem2,
                     dsem0, dsem1, dsem2):
    s = lax.axis_index("s")
    wid = s * NUM_CORES + lax.axis_index("c")
    base = wid * B_PER_W
    gsems = [gsem0, gsem1, gsem2]
    xsems = [xsem0, xsem1, xsem2]
    dsems = [dsem0, dsem1, dsem2]

    pltpu.sync_copy(idx_hbm.at[wid], idx_v)

    def start_gather(g, b):
        pltpu.make_async_copy(
            weight_hbm.at[idx_v.at[g]], rows_v.at[b], gsems[b]).start()

    def wait_gather(b):
        pltpu.make_async_copy(
            weight_hbm.at[idx_v.at[0]], rows_v.at[b], gsems[b]).wait()

    def start_xcopy(b):
        pltpu.make_async_copy(rows_v.at[b], spm.at[s, b], xsems[b]).start()

    def wait_xcopy(b):
        pltpu.make_async_copy(rows_v.at[b], spm.at[s, b], xsems[b]).wait()

    def start_drain(g, b):
        pltpu.make_async_copy(
            spm.at[s, b], out_hbm.at[pl.ds(base + g * CHUNK, CHUNK)],
            dsems[b]).start()

    def wait_drain(b):
        pltpu.make_async_copy(
            spm.at[s, b], out_hbm.at[pl.ds(base, CHUNK)], dsems[b]).wait()

    start_gather(0, 0)
    start_gather(1, 1)
    wait_gather(0)
    start_xcopy(0)
    start_gather(2, 2)
    wait_gather(1)
    start_xcopy(1)
    wait_xcopy(0)
    start_gather(3, 0)
    start_drain(0, 0)

    def steady(i, carry):
        for j in range(NBUF):
            g = 2 + NBUF * i + j
            b = (2 + j) % NBUF
            pm1 = (1 + j) % NBUF
            pm2 = j % NBUF
            wait_gather(b)
            start_xcopy(b)
            wait_xcopy(pm1)
            start_gather(g + 2, pm1)
            start_drain(g - 1, pm1)
            wait_drain(pm2)
        return carry

    lax.fori_loop(0, (NCHUNK - 5) // NBUF, steady, 0)

    g = NCHUNK - 3
    wait_gather(2)
    start_xcopy(2)
    wait_xcopy(1)
    start_gather(g + 2, 1)
    start_drain(g - 1, 1)
    wait_drain(0)
    g = NCHUNK - 2
    wait_gather(0)
    start_xcopy(0)
    wait_xcopy(2)
    start_drain(g - 1, 2)
    wait_drain(1)
    g = NCHUNK - 1
    wait_gather(1)
    start_xcopy(1)
    wait_xcopy(0)
    start_drain(g - 1, 0)
    wait_drain(2)
    wait_xcopy(1)
    start_drain(NCHUNK - 1, 1)
    wait_drain(0)
    wait_drain(1)


def kernel(embed_id, weight):
    idx = embed_id.astype(jnp.int32).reshape(NUM_WORKERS, NCHUNK, CHUNK)
    return _codebook_gather(weight, idx)
